# baseline (speedup 1.0000x reference)
import functools

import jax
import jax.numpy as jnp
from jax.experimental import pallas as pl
from jax.experimental.pallas import tpu as pltpu

HC = 128
IES = 64
E = 320000
T = 1000000

BE = 2000
BT = 4000


def _swish(v):
    return v * jax.nn.sigmoid(v)


def _pre_body(x_ref, rbf_ref, wji_ref, bji_ref, wkj_ref, bkj_ref, wrbf_ref,
              wdown_ref, xji_ref, xdown_ref):
    xb = x_ref[...]
    xji = _swish(jnp.dot(xb, wji_ref[...], preferred_element_type=jnp.float32)
                 + bji_ref[...])
    xji_ref[...] = xji
    xkj = _swish(jnp.dot(xb, wkj_ref[...], preferred_element_type=jnp.float32)
                 + bkj_ref[...])
    rbf_e = jnp.dot(rbf_ref[...], wrbf_ref[...],
                    preferred_element_type=jnp.float32)
    xkj = xkj * rbf_e
    xdown_ref[...] = _swish(
        jnp.dot(xkj, wdown_ref[...], preferred_element_type=jnp.float32))


def _sbf_body(sbf_ref, wsbf_ref, sbfe_ref):
    sbfe_ref[...] = jnp.dot(sbf_ref[...], wsbf_ref[...],
                            preferred_element_type=jnp.float32)


def _post_body(out64_ref, xji_ref, x_ref, wup_ref, wb1_ref, bb1_ref, wb2_ref,
               bb2_ref, wlin_ref, blin_ref, wa11_ref, ba11_ref, wa12_ref,
               ba12_ref, wa21_ref, ba21_ref, wa22_ref, ba22_ref, out_ref):
    xkj = _swish(jnp.dot(out64_ref[...], wup_ref[...],
                         preferred_element_type=jnp.float32))
    h = xji_ref[...] + xkj
    h = h + _swish(
        jnp.dot(_swish(jnp.dot(h, wb1_ref[...],
                               preferred_element_type=jnp.float32)
                       + bb1_ref[...]),
                wb2_ref[...], preferred_element_type=jnp.float32)
        + bb2_ref[...])
    h = _swish(jnp.dot(h, wlin_ref[...], preferred_element_type=jnp.float32)
               + blin_ref[...]) + x_ref[...]
    h = h + _swish(
        jnp.dot(_swish(jnp.dot(h, wa11_ref[...],
                               preferred_element_type=jnp.float32)
                       + ba11_ref[...]),
                wa12_ref[...], preferred_element_type=jnp.float32)
        + ba12_ref[...])
    h = h + _swish(
        jnp.dot(_swish(jnp.dot(h, wa21_ref[...],
                               preferred_element_type=jnp.float32)
                       + ba21_ref[...]),
                wa22_ref[...], preferred_element_type=jnp.float32)
        + ba22_ref[...])
    out_ref[...] = h


def _row_spec(block_rows, cols):
    return pl.BlockSpec((block_rows, cols), lambda i: (i, 0))


def _whole(shape):
    return pl.BlockSpec(shape, lambda i: tuple(0 for _ in shape))


def _pre_stage(x, rbf, p):
    wrbf = p['W_rbf1'] @ p['W_rbf2']
    grid = (E // BE,)
    return pl.pallas_call(
        _pre_body,
        grid=grid,
        in_specs=[
            _row_spec(BE, HC),
            _row_spec(BE, rbf.shape[1]),
            _whole((HC, HC)), _whole((1, HC)),
            _whole((HC, HC)), _whole((1, HC)),
            _whole((rbf.shape[1], HC)),
            _whole((HC, IES)),
        ],
        out_specs=[_row_spec(BE, HC), _row_spec(BE, IES)],
        out_shape=[jax.ShapeDtypeStruct((E, HC), jnp.float32),
                   jax.ShapeDtypeStruct((E, IES), jnp.float32)],
    )(x, rbf, p['W_ji'], p['b_ji'].reshape(1, HC), p['W_kj'],
      p['b_kj'].reshape(1, HC), wrbf, p['W_down'])


def _sbf_stage(sbf, p):
    wsbf = p['W_sbf1'] @ p['W_sbf2']
    grid = (T // BT,)
    return pl.pallas_call(
        _sbf_body,
        grid=grid,
        in_specs=[_row_spec(BT, sbf.shape[1]), _whole(wsbf.shape)],
        out_specs=_row_spec(BT, IES),
        out_shape=jax.ShapeDtypeStruct((T, IES), jnp.float32),
    )(sbf, wsbf)


def _post_stage(out64, xji, x, p):
    (wb1, bb1, wb2, bb2) = p['before'][0]
    (wa11, ba11, wa12, ba12) = p['after'][0]
    (wa21, ba21, wa22, ba22) = p['after'][1]
    grid = (E // BE,)
    return pl.pallas_call(
        _post_body,
        grid=grid,
        in_specs=[
            _row_spec(BE, IES), _row_spec(BE, HC), _row_spec(BE, HC),
            _whole((IES, HC)),
            _whole((HC, HC)), _whole((1, HC)), _whole((HC, HC)), _whole((1, HC)),
            _whole((HC, HC)), _whole((1, HC)),
            _whole((HC, HC)), _whole((1, HC)), _whole((HC, HC)), _whole((1, HC)),
            _whole((HC, HC)), _whole((1, HC)), _whole((HC, HC)), _whole((1, HC)),
        ],
        out_specs=_row_spec(BE, HC),
        out_shape=jax.ShapeDtypeStruct((E, HC), jnp.float32),
    )(out64, xji, x, p['W_up'],
      wb1, bb1.reshape(1, HC), wb2, bb2.reshape(1, HC),
      p['W_lin'], p['b_lin'].reshape(1, HC),
      wa11, ba11.reshape(1, HC), wa12, ba12.reshape(1, HC),
      wa21, ba21.reshape(1, HC), wa22, ba22.reshape(1, HC))


def kernel(x, rbf, sbf, params, idx_kj, idx_ji):
    xji, xdown = _pre_stage(x, rbf, params)
    sbfe = _sbf_stage(sbf, params)
    msg = jnp.take(xdown, idx_kj, axis=0) * sbfe
    out64 = jax.ops.segment_sum(msg, idx_ji, num_segments=E)
    return _post_stage(out64, xji, x, params)
